# Initial kernel scaffold; baseline (speedup 1.0000x reference)
import functools

import jax
import jax.numpy as jnp
from jax import lax
from jax.experimental import pallas as pl
from jax.experimental.pallas import tpu as pltpu
from jax.experimental.pallas import tpu_sc as plsc

N = 10000
T = 20
E = 160000
B = 1024
HG = 64
HT = 128

NPAD = 10240
BA = 1024
NC, NS = 2, 16
NW = NC * NS
EC = 128
KCH = -(-E // (NW * EC))
EPAD = NW * EC * KCH
QW = B // NW
QT = B // NS
RPT = NPAD // NS


def _cell(gi, gh, h, H):
    r = jax.nn.sigmoid(gi[:, :H] + gh[:, :H])
    z = jax.nn.sigmoid(gi[:, H:2 * H] + gh[:, H:2 * H])
    n = jnp.tanh(gi[:, 2 * H:] + r * gh[:, 2 * H:])
    return (1.0 - z) * n + z * h


def _dot(a, b):
    return jnp.dot(a, b, preferred_element_type=jnp.float32)


def _na_body(x_ref, wih_ref, whh_ref, bih_ref, bhh_ref, wmsg_ref, bmsg_ref,
             tf_ref, m_ref):
    xb = x_ref[...]
    wih = wih_ref[...]
    whh = whh_ref[...]
    bih = bih_ref[...]
    bhh = bhh_ref[...]
    h = jnp.zeros((BA, HG), jnp.float32)
    for t in range(T):
        xt = xb[:, 3 * t:3 * t + 3]
        gi = _dot(xt, wih) + bih
        gh = _dot(h, whh) + bhh
        h = _cell(gi, gh, h, HG)
    tf_ref[...] = h
    m_ref[...] = _dot(h, wmsg_ref[...]) + bmsg_ref[...]


def _na_encode(xp, wih_t, whh_t, bih, bhh, wmsg_t, bmsg):
    nblk = NPAD // BA
    full = lambda s: pl.BlockSpec(s, lambda i: (0, 0))
    return pl.pallas_call(
        _na_body,
        grid=(nblk,),
        in_specs=[
            pl.BlockSpec((BA, 64), lambda i: (i, 0)),
            full(wih_t.shape), full(whh_t.shape), full(bih.shape),
            full(bhh.shape), full(wmsg_t.shape), full(bmsg.shape),
        ],
        out_specs=[
            pl.BlockSpec((BA, HG), lambda i: (i, 0)),
            pl.BlockSpec((BA, HG), lambda i: (i, 0)),
        ],
        out_shape=[
            jax.ShapeDtypeStruct((NPAD, HG), jnp.float32),
            jax.ShapeDtypeStruct((NPAD, HG), jnp.float32),
        ],
    )(xp, wih_t, whh_t, bih, bhh, wmsg_t, bmsg)


def _ta_body(xq_ref, xb_ref, w0i_ref, w0h_ref, b0i_ref, b0h_ref,
             w1i_ref, w1h_ref, b1i_ref, b1h_ref, out_ref):
    xq = xq_ref[...]
    xb = xb_ref[...]
    w0i = w0i_ref[...]
    w0h = w0h_ref[...]
    w1i = w1i_ref[...]
    w1h = w1h_ref[...]
    b0i = b0i_ref[...]
    b0h = b0h_ref[...]
    b1i = b1i_ref[...]
    b1h = b1h_ref[...]
    h0 = jnp.zeros((B, HT), jnp.float32)
    h1 = jnp.zeros((B, HT), jnp.float32)
    for t in range(T):
        ballt = jnp.broadcast_to(xb[0:1, 3 * t:3 * t + 3], (B, 3))
        xt = xq[:, 3 * t:3 * t + 3]
        u = jnp.concatenate([ballt, xt], axis=1)
        h0 = _cell(_dot(u, w0i) + b0i, _dot(h0, w0h) + b0h, h0, HT)
        h1 = _cell(_dot(h0, w1i) + b1i, _dot(h1, w1h) + b1h, h1, HT)
    out_ref[...] = h1


def _ta_encode(xq, xb, w0i, w0h, b0i, b0h, w1i, w1h, b1i, b1h):
    return pl.pallas_call(
        _ta_body,
        out_shape=jax.ShapeDtypeStruct((B, HT), jnp.float32),
    )(xq, xb, w0i, w0h, b0i, b0h, w1i, w1h, b1i, b1h)


def _upd_body(p0_ref, p1_ref, h_ref, wih_ref, whh_ref, bih_ref, bhh_ref,
              wmsg_ref, bmsg_ref, h1_ref, m2_ref):
    agg = p0_ref[...] + p1_ref[...]
    h = h_ref[...]
    gi = _dot(agg, wih_ref[...]) + bih_ref[...]
    gh = _dot(h, whh_ref[...]) + bhh_ref[...]
    hn = _cell(gi, gh, h, HG)
    h1_ref[...] = hn
    m2_ref[...] = _dot(hn, wmsg_ref[...]) + bmsg_ref[...]


def _gg_update(p0, p1, h, wih_t, whh_t, bih, bhh, wmsg_t, bmsg):
    nblk = NPAD // BA
    full = lambda s: pl.BlockSpec(s, lambda i: (0, 0))
    blk = pl.BlockSpec((BA, HG), lambda i: (i, 0))
    return pl.pallas_call(
        _upd_body,
        grid=(nblk,),
        in_specs=[blk, blk, blk, full(wih_t.shape), full(whh_t.shape),
                  full(bih.shape), full(bhh.shape), full(wmsg_t.shape),
                  full(bmsg.shape)],
        out_specs=[blk, blk],
        out_shape=[
            jax.ShapeDtypeStruct((NPAD, HG), jnp.float32),
            jax.ShapeDtypeStruct((NPAD, HG), jnp.float32),
        ],
    )(p0, p1, h, wih_t, whh_t, bih, bhh, wmsg_t, bmsg)


def _head_body(pq0_ref, pq1_ref, hq_ref, tf2_ref, wih_ref, whh_ref, bih_ref,
               bhh_ref, w1_ref, b1_ref, w2_ref, b2_ref, out_ref):
    aggq = pq0_ref[...] + pq1_ref[...]
    hq = hq_ref[...]
    gi = _dot(aggq, wih_ref[...]) + bih_ref[...]
    gh = _dot(hq, whh_ref[...]) + bhh_ref[...]
    h2 = _cell(gi, gh, hq, HG)
    w1 = w1_ref[...]
    hid = jax.nn.relu(_dot(h2, w1[:HG, :]) + _dot(tf2_ref[...], w1[HG:, :])
                      + b1_ref[...])
    out_ref[...] = jax.nn.sigmoid(_dot(hid, w2_ref[...]) + b2_ref[...])


def _head(pq0, pq1, hq, tf2, wih_t, whh_t, bih, bhh, w1, b1, w2, b2):
    return pl.pallas_call(
        _head_body,
        out_shape=jax.ShapeDtypeStruct((B, 1), jnp.float32),
    )(pq0, pq1, hq, tf2, wih_t, whh_t, bih, bhh, w1, b1, w2, b2)


def _sc_mesh():
    return plsc.VectorSubcoreMesh(core_axis_name="c", subcore_axis_name="s")


def _scatter_chunks(m_hbm, src_hbm, dst_hbm, agg, eidx, didx, erows, sem,
                    wid):
    ebase = wid * (KCH * EC)

    def chunk(j, carry):
        off = ebase + j * EC
        pltpu.sync_copy(src_hbm.at[pl.ds(off, EC)], eidx)
        pltpu.async_copy(m_hbm.at[eidx], erows, sem).wait()
        pltpu.sync_copy(dst_hbm.at[pl.ds(off, EC)], didx)
        pltpu.sync_copy(erows, agg.at[didx], add=True)
        return carry

    lax.fori_loop(0, KCH, chunk, 0)


def _gather_rows(tbl_hbm, q_hbm, out_hbm, qidx, qrows, sem, base, cnt):
    pltpu.sync_copy(q_hbm.at[pl.ds(base, cnt)], qidx)
    pltpu.async_copy(tbl_hbm.at[qidx], qrows, sem).wait()
    pltpu.sync_copy(qrows, out_hbm.at[pl.ds(base, cnt)])


def _sc_step1(m1, srcp, dstp, q, xp, zrows):
    @functools.partial(
        pl.kernel,
        out_type=[
            jax.ShapeDtypeStruct((NPAD, HG), jnp.float32),
            jax.ShapeDtypeStruct((NPAD, HG), jnp.float32),
            jax.ShapeDtypeStruct((B, 64), jnp.float32),
        ],
        mesh=_sc_mesh(),
        scratch_types=[
            pltpu.VMEM((EC,), jnp.int32),
            pltpu.VMEM((EC,), jnp.int32),
            pltpu.VMEM((EC, HG), jnp.float32),
            pltpu.VMEM((QW,), jnp.int32),
            pltpu.VMEM((QW, 64), jnp.float32),
            pltpu.VMEM_SHARED((NPAD, HG), jnp.float32),
            pltpu.SemaphoreType.DMA,
        ],
    )
    def k(m_hbm, src_hbm, dst_hbm, q_hbm, xp_hbm, z_hbm,
          p0_hbm, p1_hbm, xq_hbm,
          eidx, didx, erows, qidx, qrows, agg, sem):
        c = lax.axis_index("c")
        s = lax.axis_index("s")
        wid = s * NC + c
        pltpu.sync_copy(z_hbm.at[pl.ds(s * RPT, RPT)],
                        agg.at[pl.ds(s * RPT, RPT)])
        plsc.subcore_barrier()
        _scatter_chunks(m_hbm, src_hbm, dst_hbm, agg, eidx, didx, erows, sem,
                        wid)
        plsc.subcore_barrier()

        @pl.when(c == 0)
        def _():
            pltpu.sync_copy(agg.at[pl.ds(s * RPT, RPT)],
                            p0_hbm.at[pl.ds(s * RPT, RPT)])

        @pl.when(c == 1)
        def _():
            pltpu.sync_copy(agg.at[pl.ds(s * RPT, RPT)],
                            p1_hbm.at[pl.ds(s * RPT, RPT)])

        _gather_rows(xp_hbm, q_hbm, xq_hbm, qidx, qrows, sem, wid * QW, QW)

    return k(m1, srcp, dstp, q, xp, zrows)


def _sc_step2(m2, srcp, dstp, q, h1, zrows):
    @functools.partial(
        pl.kernel,
        out_type=[
            jax.ShapeDtypeStruct((NPAD, HG), jnp.float32),
            jax.ShapeDtypeStruct((NPAD, HG), jnp.float32),
            jax.ShapeDtypeStruct((B, HG), jnp.float32),
            jax.ShapeDtypeStruct((B, HG), jnp.float32),
            jax.ShapeDtypeStruct((B, HG), jnp.float32),
        ],
        mesh=_sc_mesh(),
        scratch_types=[
            pltpu.VMEM((EC,), jnp.int32),
            pltpu.VMEM((EC,), jnp.int32),
            pltpu.VMEM((EC, HG), jnp.float32),
            pltpu.VMEM((QW,), jnp.int32),
            pltpu.VMEM((QW, HG), jnp.float32),
            pltpu.VMEM((QT,), jnp.int32),
            pltpu.VMEM((QT, HG), jnp.float32),
            pltpu.VMEM_SHARED((NPAD, HG), jnp.float32),
            pltpu.SemaphoreType.DMA,
        ],
    )
    def k(m_hbm, src_hbm, dst_hbm, q_hbm, h1_hbm, z_hbm,
          p0_hbm, p1_hbm, hq_hbm, pq0_hbm, pq1_hbm,
          eidx, didx, erows, qidx, qrows, qidx2, qrows2, agg, sem):
        c = lax.axis_index("c")
        s = lax.axis_index("s")
        wid = s * NC + c
        pltpu.sync_copy(z_hbm.at[pl.ds(s * RPT, RPT)],
                        agg.at[pl.ds(s * RPT, RPT)])
        plsc.subcore_barrier()
        _scatter_chunks(m_hbm, src_hbm, dst_hbm, agg, eidx, didx, erows, sem,
                        wid)
        plsc.subcore_barrier()

        @pl.when(c == 0)
        def _():
            pltpu.sync_copy(agg.at[pl.ds(s * RPT, RPT)],
                            p0_hbm.at[pl.ds(s * RPT, RPT)])

        @pl.when(c == 1)
        def _():
            pltpu.sync_copy(agg.at[pl.ds(s * RPT, RPT)],
                            p1_hbm.at[pl.ds(s * RPT, RPT)])

        _gather_rows(h1_hbm, q_hbm, hq_hbm, qidx, qrows, sem, wid * QW, QW)
        plsc.subcore_barrier()
        pltpu.sync_copy(q_hbm.at[pl.ds(s * QT, QT)], qidx2)

        @pl.when(c == 0)
        def _():
            pltpu.async_copy(p0_hbm.at[qidx2], qrows2, sem).wait()
            pltpu.sync_copy(qrows2, pq0_hbm.at[pl.ds(s * QT, QT)])

        @pl.when(c == 1)
        def _():
            pltpu.async_copy(p1_hbm.at[qidx2], qrows2, sem).wait()
            pltpu.sync_copy(qrows2, pq1_hbm.at[pl.ds(s * QT, QT)])

    return k(m2, srcp, dstp, q, h1, zrows)


def kernel(g, x, q_from, na_Wih, na_Whh, na_bih, na_bhh, ta0_Wih, ta0_Whh,
           ta0_bih, ta0_bhh, ta1_Wih, ta1_Whh, ta1_bih, ta1_bhh, gg_Wmsg,
           gg_bmsg, gg_Wih, gg_Whh, gg_bih, gg_bhh, p_W1, p_b1, p_W2, p_b2):
    f32 = jnp.float32
    i32 = jnp.int32

    x_flat = x.reshape(N, T * 3).astype(f32)
    xp = jnp.pad(x_flat, ((0, NPAD - N), (0, 64 - T * 3)))
    xb = xp[0:1]
    src = jnp.concatenate([g[0].astype(i32),
                           jnp.full((EPAD - E,), NPAD - 1, i32)])
    dst = jnp.concatenate([g[1].astype(i32),
                           jnp.full((EPAD - E,), NPAD - 1, i32)])
    q = q_from.astype(i32)
    zrows = jnp.zeros((NPAD, HG), f32)
    row = lambda b: b.reshape(1, -1).astype(f32)

    tf, m1 = _na_encode(xp, na_Wih.T, na_Whh.T, row(na_bih), row(na_bhh),
                        gg_Wmsg.T, row(gg_bmsg))

    p0, p1, xq = _sc_step1(m1, src, dst, q, xp, zrows)

    tf2q = _ta_encode(xq, xb, ta0_Wih.T, ta0_Whh.T, row(ta0_bih),
                      row(ta0_bhh), ta1_Wih.T, ta1_Whh.T, row(ta1_bih),
                      row(ta1_bhh))

    h1, m2 = _gg_update(p0, p1, tf, gg_Wih.T, gg_Whh.T, row(gg_bih),
                        row(gg_bhh), gg_Wmsg.T, row(gg_bmsg))

    _, _, hq, pq0, pq1 = _sc_step2(m2, src, dst, q, h1, zrows)

    return _head(pq0, pq1, hq, tf2q, gg_Wih.T, gg_Whh.T, row(gg_bih),
                 row(gg_bhh), p_W1.astype(f32), row(p_b1),
                 p_W2.astype(f32), row(p_b2))


# speedup vs baseline: 4.2624x; 4.2624x over previous
import functools

import jax
import jax.numpy as jnp
from jax import lax
from jax.experimental import pallas as pl
from jax.experimental.pallas import tpu as pltpu
from jax.experimental.pallas import tpu_sc as plsc

N = 10000
T = 20
E = 160000
B = 1024
HG = 64
HT = 128

NPAD = 10240
BA = 1024
NC, NS = 2, 16
NW = NC * NS
EC = 128
KCH = -(-E // (NW * EC))
EPAD = NW * EC * KCH
QW = B // NW
QT = B // NS
RPT = NPAD // NS


def _cell(gi, gh, h, H):
    r = jax.nn.sigmoid(gi[:, :H] + gh[:, :H])
    z = jax.nn.sigmoid(gi[:, H:2 * H] + gh[:, H:2 * H])
    n = jnp.tanh(gi[:, 2 * H:] + r * gh[:, 2 * H:])
    return (1.0 - z) * n + z * h


def _dot(a, b):
    return jnp.dot(a, b, preferred_element_type=jnp.float32)


def _na_body(x_ref, wih_ref, whh_ref, bih_ref, bhh_ref, wmsg_ref, bmsg_ref,
             tf_ref, m_ref):
    xb = x_ref[...]
    wih = wih_ref[...]
    whh = whh_ref[...]
    bih = bih_ref[...]
    bhh = bhh_ref[...]
    h = jnp.zeros((BA, HG), jnp.float32)
    for t in range(T):
        xt = xb[:, 3 * t:3 * t + 3]
        gi = _dot(xt, wih) + bih
        gh = _dot(h, whh) + bhh
        h = _cell(gi, gh, h, HG)
    tf_ref[...] = h
    m_ref[...] = _dot(h, wmsg_ref[...]) + bmsg_ref[...]


def _na_encode(xp, wih_t, whh_t, bih, bhh, wmsg_t, bmsg):
    nblk = NPAD // BA
    full = lambda s: pl.BlockSpec(s, lambda i: (0, 0))
    return pl.pallas_call(
        _na_body,
        grid=(nblk,),
        in_specs=[
            pl.BlockSpec((BA, 64), lambda i: (i, 0)),
            full(wih_t.shape), full(whh_t.shape), full(bih.shape),
            full(bhh.shape), full(wmsg_t.shape), full(bmsg.shape),
        ],
        out_specs=[
            pl.BlockSpec((BA, HG), lambda i: (i, 0)),
            pl.BlockSpec((BA, HG), lambda i: (i, 0)),
        ],
        out_shape=[
            jax.ShapeDtypeStruct((NPAD, HG), jnp.float32),
            jax.ShapeDtypeStruct((NPAD, HG), jnp.float32),
        ],
    )(xp, wih_t, whh_t, bih, bhh, wmsg_t, bmsg)


def _ta_body(xq_ref, xb_ref, w0i_ref, w0h_ref, b0i_ref, b0h_ref,
             w1i_ref, w1h_ref, b1i_ref, b1h_ref, out_ref):
    xq = xq_ref[...]
    xb = xb_ref[...]
    w0i = w0i_ref[...]
    w0h = w0h_ref[...]
    w1i = w1i_ref[...]
    w1h = w1h_ref[...]
    b0i = b0i_ref[...]
    b0h = b0h_ref[...]
    b1i = b1i_ref[...]
    b1h = b1h_ref[...]
    h0 = jnp.zeros((B, HT), jnp.float32)
    h1 = jnp.zeros((B, HT), jnp.float32)
    for t in range(T):
        ballt = jnp.broadcast_to(xb[0:1, 3 * t:3 * t + 3], (B, 3))
        xt = xq[:, 3 * t:3 * t + 3]
        u = jnp.concatenate([ballt, xt], axis=1)
        h0 = _cell(_dot(u, w0i) + b0i, _dot(h0, w0h) + b0h, h0, HT)
        h1 = _cell(_dot(h0, w1i) + b1i, _dot(h1, w1h) + b1h, h1, HT)
    out_ref[...] = h1


def _ta_encode(xq, xb, w0i, w0h, b0i, b0h, w1i, w1h, b1i, b1h):
    return pl.pallas_call(
        _ta_body,
        out_shape=jax.ShapeDtypeStruct((B, HT), jnp.float32),
    )(xq, xb, w0i, w0h, b0i, b0h, w1i, w1h, b1i, b1h)


def _upd_body(p0_ref, p1_ref, h_ref, wih_ref, whh_ref, bih_ref, bhh_ref,
              wmsg_ref, bmsg_ref, h1_ref, m2_ref):
    agg = p0_ref[...] + p1_ref[...]
    h = h_ref[...]
    gi = _dot(agg, wih_ref[...]) + bih_ref[...]
    gh = _dot(h, whh_ref[...]) + bhh_ref[...]
    hn = _cell(gi, gh, h, HG)
    h1_ref[...] = hn
    m2_ref[...] = _dot(hn, wmsg_ref[...]) + bmsg_ref[...]


def _gg_update(p0, p1, h, wih_t, whh_t, bih, bhh, wmsg_t, bmsg):
    nblk = NPAD // BA
    full = lambda s: pl.BlockSpec(s, lambda i: (0, 0))
    blk = pl.BlockSpec((BA, HG), lambda i: (i, 0))
    return pl.pallas_call(
        _upd_body,
        grid=(nblk,),
        in_specs=[blk, blk, blk, full(wih_t.shape), full(whh_t.shape),
                  full(bih.shape), full(bhh.shape), full(wmsg_t.shape),
                  full(bmsg.shape)],
        out_specs=[blk, blk],
        out_shape=[
            jax.ShapeDtypeStruct((NPAD, HG), jnp.float32),
            jax.ShapeDtypeStruct((NPAD, HG), jnp.float32),
        ],
    )(p0, p1, h, wih_t, whh_t, bih, bhh, wmsg_t, bmsg)


def _head_body(pq0_ref, pq1_ref, hq_ref, tf2_ref, wih_ref, whh_ref, bih_ref,
               bhh_ref, w1_ref, b1_ref, w2_ref, b2_ref, out_ref):
    aggq = pq0_ref[...] + pq1_ref[...]
    hq = hq_ref[...]
    gi = _dot(aggq, wih_ref[...]) + bih_ref[...]
    gh = _dot(hq, whh_ref[...]) + bhh_ref[...]
    h2 = _cell(gi, gh, hq, HG)
    w1 = w1_ref[...]
    hid = jax.nn.relu(_dot(h2, w1[:HG, :]) + _dot(tf2_ref[...], w1[HG:, :])
                      + b1_ref[...])
    out_ref[...] = jax.nn.sigmoid(_dot(hid, w2_ref[...]) + b2_ref[...])


def _head(pq0, pq1, hq, tf2, wih_t, whh_t, bih, bhh, w1, b1, w2, b2):
    return pl.pallas_call(
        _head_body,
        out_shape=jax.ShapeDtypeStruct((B, 1), jnp.float32),
    )(pq0, pq1, hq, tf2, wih_t, whh_t, bih, bhh, w1, b1, w2, b2)


def _sc_mesh():
    return plsc.VectorSubcoreMesh(core_axis_name="c", subcore_axis_name="s")


_SC_PARAMS = pltpu.CompilerParams(use_tc_tiling_on_sc=False)


def _scatter_chunks(m_hbm, src_hbm, dst_hbm, agg, eidx, didx, erows, sem,
                    wid):
    ebase = wid * (KCH * EC)

    def chunk(j, carry):
        off = ebase + j * EC
        pltpu.sync_copy(src_hbm.at[pl.ds(off, EC)], eidx)
        pltpu.async_copy(m_hbm.at[eidx], erows, sem).wait()
        pltpu.sync_copy(dst_hbm.at[pl.ds(off, EC)], didx)
        pltpu.sync_copy(erows, agg.at[didx], add=True)
        return carry

    lax.fori_loop(0, KCH, chunk, 0)


def _gather_rows(tbl_hbm, q_hbm, out_hbm, qidx, qrows, sem, base, cnt):
    pltpu.sync_copy(q_hbm.at[pl.ds(base, cnt)], qidx)
    pltpu.async_copy(tbl_hbm.at[qidx], qrows, sem).wait()
    pltpu.sync_copy(qrows, out_hbm.at[pl.ds(base, cnt)])


def _sc_step1(m1, srcp, dstp, q, xp, zrows):
    @functools.partial(
        pl.kernel,
        out_type=[
            jax.ShapeDtypeStruct((NPAD, HG), jnp.float32),
            jax.ShapeDtypeStruct((NPAD, HG), jnp.float32),
            jax.ShapeDtypeStruct((B, 64), jnp.float32),
        ],
        mesh=_sc_mesh(),
        scratch_types=[
            pltpu.VMEM((EC,), jnp.int32),
            pltpu.VMEM((EC,), jnp.int32),
            pltpu.VMEM((EC, HG), jnp.float32),
            pltpu.VMEM((QW,), jnp.int32),
            pltpu.VMEM((QW, 64), jnp.float32),
            pltpu.VMEM_SHARED((NPAD, HG), jnp.float32),
            pltpu.SemaphoreType.DMA,
        ],
        compiler_params=_SC_PARAMS,
    )
    def k(m_hbm, src_hbm, dst_hbm, q_hbm, xp_hbm, z_hbm,
          p0_hbm, p1_hbm, xq_hbm,
          eidx, didx, erows, qidx, qrows, agg, sem):
        c = lax.axis_index("c")
        s = lax.axis_index("s")
        wid = s * NC + c
        pltpu.sync_copy(z_hbm.at[pl.ds(s * RPT, RPT)],
                        agg.at[pl.ds(s * RPT, RPT)])
        plsc.subcore_barrier()
        _scatter_chunks(m_hbm, src_hbm, dst_hbm, agg, eidx, didx, erows, sem,
                        wid)
        plsc.subcore_barrier()

        @pl.when(c == 0)
        def _():
            pltpu.sync_copy(agg.at[pl.ds(s * RPT, RPT)],
                            p0_hbm.at[pl.ds(s * RPT, RPT)])

        @pl.when(c == 1)
        def _():
            pltpu.sync_copy(agg.at[pl.ds(s * RPT, RPT)],
                            p1_hbm.at[pl.ds(s * RPT, RPT)])

        _gather_rows(xp_hbm, q_hbm, xq_hbm, qidx, qrows, sem, wid * QW, QW)

    return k(m1, srcp, dstp, q, xp, zrows)


def _sc_step2(m2, srcp, dstp, q, h1, zrows):
    @functools.partial(
        pl.kernel,
        out_type=[
            jax.ShapeDtypeStruct((NPAD, HG), jnp.float32),
            jax.ShapeDtypeStruct((NPAD, HG), jnp.float32),
            jax.ShapeDtypeStruct((B, HG), jnp.float32),
            jax.ShapeDtypeStruct((B, HG), jnp.float32),
            jax.ShapeDtypeStruct((B, HG), jnp.float32),
        ],
        mesh=_sc_mesh(),
        scratch_types=[
            pltpu.VMEM((EC,), jnp.int32),
            pltpu.VMEM((EC,), jnp.int32),
            pltpu.VMEM((EC, HG), jnp.float32),
            pltpu.VMEM((QW,), jnp.int32),
            pltpu.VMEM((QW, HG), jnp.float32),
            pltpu.VMEM((QT,), jnp.int32),
            pltpu.VMEM((QT, HG), jnp.float32),
            pltpu.VMEM_SHARED((NPAD, HG), jnp.float32),
            pltpu.SemaphoreType.DMA,
        ],
        compiler_params=_SC_PARAMS,
    )
    def k(m_hbm, src_hbm, dst_hbm, q_hbm, h1_hbm, z_hbm,
          p0_hbm, p1_hbm, hq_hbm, pq0_hbm, pq1_hbm,
          eidx, didx, erows, qidx, qrows, qidx2, qrows2, agg, sem):
        c = lax.axis_index("c")
        s = lax.axis_index("s")
        wid = s * NC + c
        pltpu.sync_copy(z_hbm.at[pl.ds(s * RPT, RPT)],
                        agg.at[pl.ds(s * RPT, RPT)])
        plsc.subcore_barrier()
        _scatter_chunks(m_hbm, src_hbm, dst_hbm, agg, eidx, didx, erows, sem,
                        wid)
        plsc.subcore_barrier()

        @pl.when(c == 0)
        def _():
            pltpu.sync_copy(agg.at[pl.ds(s * RPT, RPT)],
                            p0_hbm.at[pl.ds(s * RPT, RPT)])

        @pl.when(c == 1)
        def _():
            pltpu.sync_copy(agg.at[pl.ds(s * RPT, RPT)],
                            p1_hbm.at[pl.ds(s * RPT, RPT)])

        _gather_rows(h1_hbm, q_hbm, hq_hbm, qidx, qrows, sem, wid * QW, QW)
        plsc.subcore_barrier()
        pltpu.sync_copy(q_hbm.at[pl.ds(s * QT, QT)], qidx2)

        @pl.when(c == 0)
        def _():
            pltpu.async_copy(p0_hbm.at[qidx2], qrows2, sem).wait()
            pltpu.sync_copy(qrows2, pq0_hbm.at[pl.ds(s * QT, QT)])

        @pl.when(c == 1)
        def _():
            pltpu.async_copy(p1_hbm.at[qidx2], qrows2, sem).wait()
            pltpu.sync_copy(qrows2, pq1_hbm.at[pl.ds(s * QT, QT)])

    return k(m2, srcp, dstp, q, h1, zrows)


def kernel(g, x, q_from, na_Wih, na_Whh, na_bih, na_bhh, ta0_Wih, ta0_Whh,
           ta0_bih, ta0_bhh, ta1_Wih, ta1_Whh, ta1_bih, ta1_bhh, gg_Wmsg,
           gg_bmsg, gg_Wih, gg_Whh, gg_bih, gg_bhh, p_W1, p_b1, p_W2, p_b2):
    f32 = jnp.float32
    i32 = jnp.int32

    x_flat = x.reshape(N, T * 3).astype(f32)
    xp = jnp.pad(x_flat, ((0, NPAD - N), (0, 64 - T * 3)))
    xb = xp[0:1]
    src = jnp.concatenate([g[0].astype(i32),
                           jnp.full((EPAD - E,), NPAD - 1, i32)])
    dst = jnp.concatenate([g[1].astype(i32),
                           jnp.full((EPAD - E,), NPAD - 1, i32)])
    q = q_from.astype(i32)
    zrows = jnp.zeros((NPAD, HG), f32)
    row = lambda b: b.reshape(1, -1).astype(f32)

    tf, m1 = _na_encode(xp, na_Wih.T, na_Whh.T, row(na_bih), row(na_bhh),
                        gg_Wmsg.T, row(gg_bmsg))

    p0, p1, xq = _sc_step1(m1, src, dst, q, xp, zrows)

    tf2q = _ta_encode(xq, xb, ta0_Wih.T, ta0_Whh.T, row(ta0_bih),
                      row(ta0_bhh), ta1_Wih.T, ta1_Whh.T, row(ta1_bih),
                      row(ta1_bhh))

    h1, m2 = _gg_update(p0, p1, tf, gg_Wih.T, gg_Whh.T, row(gg_bih),
                        row(gg_bhh), gg_Wmsg.T, row(gg_bmsg))

    _, _, hq, pq0, pq1 = _sc_step2(m2, src, dst, q, h1, zrows)

    return _head(pq0, pq1, hq, tf2q, gg_Wih.T, gg_Whh.T, row(gg_bih),
                 row(gg_bhh), p_W1.astype(f32), row(p_b1),
                 p_W2.astype(f32), row(p_b2))


# speedup vs baseline: 5.0667x; 1.1887x over previous
import functools

import jax
import jax.numpy as jnp
from jax import lax
from jax.experimental import pallas as pl
from jax.experimental.pallas import tpu as pltpu
from jax.experimental.pallas import tpu_sc as plsc

N = 10000
T = 20
E = 160000
B = 1024
HG = 64
HT = 128

NPAD = 10240
BA = 1024
NC, NS = 2, 16
NW = NC * NS
EC = 128
KCH = -(-E // (NW * EC))
EPAD = NW * EC * KCH
QW = B // NW
QT = B // NS
RPT = NPAD // NS


def _cell(gi, gh, h, H):
    r = jax.nn.sigmoid(gi[:, :H] + gh[:, :H])
    z = jax.nn.sigmoid(gi[:, H:2 * H] + gh[:, H:2 * H])
    n = jnp.tanh(gi[:, 2 * H:] + r * gh[:, 2 * H:])
    return (1.0 - z) * n + z * h


def _dot(a, b):
    return jnp.dot(a, b, preferred_element_type=jnp.float32)


def _na_body(x_ref, wih_ref, whh_ref, bih_ref, bhh_ref, wmsg_ref, bmsg_ref,
             tf_ref, m_ref):
    xb = x_ref[...]
    wih = wih_ref[...]
    whh = whh_ref[...]
    bih = bih_ref[...]
    bhh = bhh_ref[...]
    h = jnp.zeros((BA, HG), jnp.float32)
    for t in range(T):
        xt = xb[:, 3 * t:3 * t + 3]
        gi = _dot(xt, wih) + bih
        gh = _dot(h, whh) + bhh
        h = _cell(gi, gh, h, HG)
    tf_ref[...] = h
    m_ref[...] = _dot(h, wmsg_ref[...]) + bmsg_ref[...]


def _na_encode(xp, wih_t, whh_t, bih, bhh, wmsg_t, bmsg):
    nblk = NPAD // BA
    full = lambda s: pl.BlockSpec(s, lambda i: (0, 0))
    return pl.pallas_call(
        _na_body,
        grid=(nblk,),
        in_specs=[
            pl.BlockSpec((BA, 64), lambda i: (i, 0)),
            full(wih_t.shape), full(whh_t.shape), full(bih.shape),
            full(bhh.shape), full(wmsg_t.shape), full(bmsg.shape),
        ],
        out_specs=[
            pl.BlockSpec((BA, HG), lambda i: (i, 0)),
            pl.BlockSpec((BA, HG), lambda i: (i, 0)),
        ],
        out_shape=[
            jax.ShapeDtypeStruct((NPAD, HG), jnp.float32),
            jax.ShapeDtypeStruct((NPAD, HG), jnp.float32),
        ],
    )(xp, wih_t, whh_t, bih, bhh, wmsg_t, bmsg)


def _ta_body(xq_ref, xb_ref, w0i_ref, w0h_ref, b0i_ref, b0h_ref,
             w1i_ref, w1h_ref, b1i_ref, b1h_ref, out_ref):
    xq = xq_ref[...]
    xb = xb_ref[...]
    w0i = w0i_ref[...]
    w0h = w0h_ref[...]
    w1i = w1i_ref[...]
    w1h = w1h_ref[...]
    b0i = b0i_ref[...]
    b0h = b0h_ref[...]
    b1i = b1i_ref[...]
    b1h = b1h_ref[...]
    h0 = jnp.zeros((B, HT), jnp.float32)
    h1 = jnp.zeros((B, HT), jnp.float32)
    for t in range(T):
        ballt = jnp.broadcast_to(xb[0:1, 3 * t:3 * t + 3], (B, 3))
        xt = xq[:, 3 * t:3 * t + 3]
        u = jnp.concatenate([ballt, xt], axis=1)
        h0 = _cell(_dot(u, w0i) + b0i, _dot(h0, w0h) + b0h, h0, HT)
        h1 = _cell(_dot(h0, w1i) + b1i, _dot(h1, w1h) + b1h, h1, HT)
    out_ref[...] = h1


def _ta_encode(xq, xb, w0i, w0h, b0i, b0h, w1i, w1h, b1i, b1h):
    return pl.pallas_call(
        _ta_body,
        out_shape=jax.ShapeDtypeStruct((B, HT), jnp.float32),
    )(xq, xb, w0i, w0h, b0i, b0h, w1i, w1h, b1i, b1h)


def _upd_body(p0_ref, p1_ref, h_ref, wih_ref, whh_ref, bih_ref, bhh_ref,
              wmsg_ref, bmsg_ref, h1_ref, m2_ref):
    agg = p0_ref[...] + p1_ref[...]
    h = h_ref[...]
    gi = _dot(agg, wih_ref[...]) + bih_ref[...]
    gh = _dot(h, whh_ref[...]) + bhh_ref[...]
    hn = _cell(gi, gh, h, HG)
    h1_ref[...] = hn
    m2_ref[...] = _dot(hn, wmsg_ref[...]) + bmsg_ref[...]


def _gg_update(p0, p1, h, wih_t, whh_t, bih, bhh, wmsg_t, bmsg):
    nblk = NPAD // BA
    full = lambda s: pl.BlockSpec(s, lambda i: (0, 0))
    blk = pl.BlockSpec((BA, HG), lambda i: (i, 0))
    return pl.pallas_call(
        _upd_body,
        grid=(nblk,),
        in_specs=[blk, blk, blk, full(wih_t.shape), full(whh_t.shape),
                  full(bih.shape), full(bhh.shape), full(wmsg_t.shape),
                  full(bmsg.shape)],
        out_specs=[blk, blk],
        out_shape=[
            jax.ShapeDtypeStruct((NPAD, HG), jnp.float32),
            jax.ShapeDtypeStruct((NPAD, HG), jnp.float32),
        ],
    )(p0, p1, h, wih_t, whh_t, bih, bhh, wmsg_t, bmsg)


def _head_body(pq0_ref, pq1_ref, hq_ref, tf2_ref, wih_ref, whh_ref, bih_ref,
               bhh_ref, w1_ref, b1_ref, w2_ref, b2_ref, out_ref):
    aggq = pq0_ref[...] + pq1_ref[...]
    hq = hq_ref[...]
    gi = _dot(aggq, wih_ref[...]) + bih_ref[...]
    gh = _dot(hq, whh_ref[...]) + bhh_ref[...]
    h2 = _cell(gi, gh, hq, HG)
    w1 = w1_ref[...]
    hid = jax.nn.relu(_dot(h2, w1[:HG, :]) + _dot(tf2_ref[...], w1[HG:, :])
                      + b1_ref[...])
    out_ref[...] = jax.nn.sigmoid(_dot(hid, w2_ref[...]) + b2_ref[...])


def _head(pq0, pq1, hq, tf2, wih_t, whh_t, bih, bhh, w1, b1, w2, b2):
    return pl.pallas_call(
        _head_body,
        out_shape=jax.ShapeDtypeStruct((B, 1), jnp.float32),
    )(pq0, pq1, hq, tf2, wih_t, whh_t, bih, bhh, w1, b1, w2, b2)


def _sc_mesh():
    return plsc.VectorSubcoreMesh(core_axis_name="c", subcore_axis_name="s")


_SC_PARAMS = pltpu.CompilerParams(use_tc_tiling_on_sc=False)


NB = 4


def _scatter_chunks(m_hbm, src_hbm, dst_hbm, agg, sidx, didx, bufs, gsems,
                    ssems, wid):
    pltpu.sync_copy(src_hbm.at[wid], sidx)
    pltpu.sync_copy(dst_hbm.at[wid], didx)

    def body(j, carry):
        gh = []
        for b in range(NB):
            jj = j * NB + b
            gh.append(pltpu.async_copy(m_hbm.at[sidx.at[jj]], bufs[b],
                                       gsems[b]))
        sh = []
        for b in range(NB):
            jj = j * NB + b
            gh[b].wait()
            sh.append(pltpu.async_copy(bufs[b], agg.at[didx.at[jj]],
                                       ssems[b], add=True))
        for b in range(NB):
            sh[b].wait()
        return carry

    lax.fori_loop(0, KCH // NB, body, 0)


def _gather_rows(tbl_hbm, q_hbm, out_hbm, qidx, qrows, sem, base, cnt):
    pltpu.sync_copy(q_hbm.at[pl.ds(base, cnt)], qidx)
    pltpu.async_copy(tbl_hbm.at[qidx], qrows, sem).wait()
    pltpu.sync_copy(qrows, out_hbm.at[pl.ds(base, cnt)])


def _sc_step1(m1, srcp, dstp, q, xp, zrows):
    @functools.partial(
        pl.kernel,
        out_type=[
            jax.ShapeDtypeStruct((NPAD, HG), jnp.float32),
            jax.ShapeDtypeStruct((NPAD, HG), jnp.float32),
            jax.ShapeDtypeStruct((B, 64), jnp.float32),
        ],
        mesh=_sc_mesh(),
        scratch_types=[
            pltpu.VMEM((KCH, EC), jnp.int32),
            pltpu.VMEM((KCH, EC), jnp.int32),
            [pltpu.VMEM((EC, HG), jnp.float32)] * NB,
            [pltpu.SemaphoreType.DMA] * NB,
            [pltpu.SemaphoreType.DMA] * NB,
            pltpu.VMEM((QW,), jnp.int32),
            pltpu.VMEM((QW, 64), jnp.float32),
            pltpu.VMEM_SHARED((NPAD, HG), jnp.float32),
            pltpu.SemaphoreType.DMA,
        ],
        compiler_params=_SC_PARAMS,
    )
    def k(m_hbm, src_hbm, dst_hbm, q_hbm, xp_hbm, z_hbm,
          p0_hbm, p1_hbm, xq_hbm,
          sidx, didx, bufs, gsems, ssems, qidx, qrows, agg, sem):
        c = lax.axis_index("c")
        s = lax.axis_index("s")
        wid = s * NC + c
        pltpu.sync_copy(z_hbm.at[pl.ds(s * RPT, RPT)],
                        agg.at[pl.ds(s * RPT, RPT)])
        plsc.subcore_barrier()
        _scatter_chunks(m_hbm, src_hbm, dst_hbm, agg, sidx, didx, bufs,
                        gsems, ssems, wid)
        plsc.subcore_barrier()

        @pl.when(c == 0)
        def _():
            pltpu.sync_copy(agg.at[pl.ds(s * RPT, RPT)],
                            p0_hbm.at[pl.ds(s * RPT, RPT)])

        @pl.when(c == 1)
        def _():
            pltpu.sync_copy(agg.at[pl.ds(s * RPT, RPT)],
                            p1_hbm.at[pl.ds(s * RPT, RPT)])

        _gather_rows(xp_hbm, q_hbm, xq_hbm, qidx, qrows, sem, wid * QW, QW)

    return k(m1, srcp, dstp, q, xp, zrows)


def _sc_step2(m2, srcp, dstp, q, h1, zrows):
    @functools.partial(
        pl.kernel,
        out_type=[
            jax.ShapeDtypeStruct((NPAD, HG), jnp.float32),
            jax.ShapeDtypeStruct((NPAD, HG), jnp.float32),
            jax.ShapeDtypeStruct((B, HG), jnp.float32),
            jax.ShapeDtypeStruct((B, HG), jnp.float32),
            jax.ShapeDtypeStruct((B, HG), jnp.float32),
        ],
        mesh=_sc_mesh(),
        scratch_types=[
            pltpu.VMEM((KCH, EC), jnp.int32),
            pltpu.VMEM((KCH, EC), jnp.int32),
            [pltpu.VMEM((EC, HG), jnp.float32)] * NB,
            [pltpu.SemaphoreType.DMA] * NB,
            [pltpu.SemaphoreType.DMA] * NB,
            pltpu.VMEM((QW,), jnp.int32),
            pltpu.VMEM((QW, HG), jnp.float32),
            pltpu.VMEM((QT,), jnp.int32),
            pltpu.VMEM((QT, HG), jnp.float32),
            pltpu.VMEM_SHARED((NPAD, HG), jnp.float32),
            pltpu.SemaphoreType.DMA,
        ],
        compiler_params=_SC_PARAMS,
    )
    def k(m_hbm, src_hbm, dst_hbm, q_hbm, h1_hbm, z_hbm,
          p0_hbm, p1_hbm, hq_hbm, pq0_hbm, pq1_hbm,
          sidx, didx, bufs, gsems, ssems, qidx, qrows, qidx2, qrows2,
          agg, sem):
        c = lax.axis_index("c")
        s = lax.axis_index("s")
        wid = s * NC + c
        pltpu.sync_copy(z_hbm.at[pl.ds(s * RPT, RPT)],
                        agg.at[pl.ds(s * RPT, RPT)])
        plsc.subcore_barrier()
        _scatter_chunks(m_hbm, src_hbm, dst_hbm, agg, sidx, didx, bufs,
                        gsems, ssems, wid)
        plsc.subcore_barrier()

        @pl.when(c == 0)
        def _():
            pltpu.sync_copy(agg.at[pl.ds(s * RPT, RPT)],
                            p0_hbm.at[pl.ds(s * RPT, RPT)])

        @pl.when(c == 1)
        def _():
            pltpu.sync_copy(agg.at[pl.ds(s * RPT, RPT)],
                            p1_hbm.at[pl.ds(s * RPT, RPT)])

        _gather_rows(h1_hbm, q_hbm, hq_hbm, qidx, qrows, sem, wid * QW, QW)
        plsc.subcore_barrier()
        pltpu.sync_copy(q_hbm.at[pl.ds(s * QT, QT)], qidx2)

        @pl.when(c == 0)
        def _():
            pltpu.async_copy(p0_hbm.at[qidx2], qrows2, sem).wait()
            pltpu.sync_copy(qrows2, pq0_hbm.at[pl.ds(s * QT, QT)])

        @pl.when(c == 1)
        def _():
            pltpu.async_copy(p1_hbm.at[qidx2], qrows2, sem).wait()
            pltpu.sync_copy(qrows2, pq1_hbm.at[pl.ds(s * QT, QT)])

    return k(m2, srcp, dstp, q, h1, zrows)


def kernel(g, x, q_from, na_Wih, na_Whh, na_bih, na_bhh, ta0_Wih, ta0_Whh,
           ta0_bih, ta0_bhh, ta1_Wih, ta1_Whh, ta1_bih, ta1_bhh, gg_Wmsg,
           gg_bmsg, gg_Wih, gg_Whh, gg_bih, gg_bhh, p_W1, p_b1, p_W2, p_b2):
    f32 = jnp.float32
    i32 = jnp.int32

    x_flat = x.reshape(N, T * 3).astype(f32)
    xp = jnp.pad(x_flat, ((0, NPAD - N), (0, 64 - T * 3)))
    xb = xp[0:1]
    src = jnp.concatenate([g[0].astype(i32),
                           jnp.full((EPAD - E,), NPAD - 1, i32)])
    src = src.reshape(NW, KCH, EC)
    dst = jnp.concatenate([g[1].astype(i32),
                           jnp.full((EPAD - E,), NPAD - 1, i32)])
    dst = dst.reshape(NW, KCH, EC)
    q = q_from.astype(i32)
    zrows = jnp.zeros((NPAD, HG), f32)
    row = lambda b: b.reshape(1, -1).astype(f32)

    tf, m1 = _na_encode(xp, na_Wih.T, na_Whh.T, row(na_bih), row(na_bhh),
                        gg_Wmsg.T, row(gg_bmsg))

    p0, p1, xq = _sc_step1(m1, src, dst, q, xp, zrows)

    tf2q = _ta_encode(xq, xb, ta0_Wih.T, ta0_Whh.T, row(ta0_bih),
                      row(ta0_bhh), ta1_Wih.T, ta1_Whh.T, row(ta1_bih),
                      row(ta1_bhh))

    h1, m2 = _gg_update(p0, p1, tf, gg_Wih.T, gg_Whh.T, row(gg_bih),
                        row(gg_bhh), gg_Wmsg.T, row(gg_bmsg))

    _, _, hq, pq0, pq1 = _sc_step2(m2, src, dst, q, h1, zrows)

    return _head(pq0, pq1, hq, tf2q, gg_Wih.T, gg_Whh.T, row(gg_bih),
                 row(gg_bhh), p_W1.astype(f32), row(p_b1),
                 p_W2.astype(f32), row(p_b2))


# speedup vs baseline: 7.3080x; 1.4424x over previous
import functools

import jax
import jax.numpy as jnp
from jax import lax
from jax.experimental import pallas as pl
from jax.experimental.pallas import tpu as pltpu
from jax.experimental.pallas import tpu_sc as plsc

N = 10000
T = 20
E = 160000
B = 1024
HG = 64
HT = 128

NPAD = 10240
BA = 1024
NC, NS = 2, 16
NW = NC * NS
EC = 128
KCH = -(-E // (NW * EC))
EPAD = NW * EC * KCH
QW = B // NW
QT = B // NS
RPT = NPAD // NS


def _cell(gi, gh, h, H):
    r = jax.nn.sigmoid(gi[:, :H] + gh[:, :H])
    z = jax.nn.sigmoid(gi[:, H:2 * H] + gh[:, H:2 * H])
    n = jnp.tanh(gi[:, 2 * H:] + r * gh[:, 2 * H:])
    return (1.0 - z) * n + z * h


def _dot(a, b):
    return jnp.dot(a, b, preferred_element_type=jnp.float32)


def _na_body(x_ref, wih_ref, whh_ref, bih_ref, bhh_ref, wmsg_ref, bmsg_ref,
             tf_ref, m_ref):
    xb = x_ref[...]
    wih = wih_ref[...]
    whh = whh_ref[...]
    bih = bih_ref[...]
    bhh = bhh_ref[...]
    h = jnp.zeros((BA, HG), jnp.float32)
    for t in range(T):
        xt = xb[:, 3 * t:3 * t + 3]
        gi = _dot(xt, wih) + bih
        gh = _dot(h, whh) + bhh
        h = _cell(gi, gh, h, HG)
    tf_ref[...] = h
    m_ref[...] = _dot(h, wmsg_ref[...]) + bmsg_ref[...]


def _na_encode(xp, wih_t, whh_t, bih, bhh, wmsg_t, bmsg):
    nblk = NPAD // BA
    full = lambda s: pl.BlockSpec(s, lambda i: (0, 0))
    return pl.pallas_call(
        _na_body,
        grid=(nblk,),
        in_specs=[
            pl.BlockSpec((BA, 64), lambda i: (i, 0)),
            full(wih_t.shape), full(whh_t.shape), full(bih.shape),
            full(bhh.shape), full(wmsg_t.shape), full(bmsg.shape),
        ],
        out_specs=[
            pl.BlockSpec((BA, HG), lambda i: (i, 0)),
            pl.BlockSpec((BA, HG), lambda i: (i, 0)),
        ],
        out_shape=[
            jax.ShapeDtypeStruct((NPAD, HG), jnp.float32),
            jax.ShapeDtypeStruct((NPAD, HG), jnp.float32),
        ],
    )(xp, wih_t, whh_t, bih, bhh, wmsg_t, bmsg)


def _ta_body(xq_ref, xb_ref, w0i_ref, w0h_ref, b0i_ref, b0h_ref,
             w1i_ref, w1h_ref, b1i_ref, b1h_ref, out_ref):
    xq = xq_ref[...]
    xb = xb_ref[...]
    w0i = w0i_ref[...]
    w0h = w0h_ref[...]
    w1i = w1i_ref[...]
    w1h = w1h_ref[...]
    b0i = b0i_ref[...]
    b0h = b0h_ref[...]
    b1i = b1i_ref[...]
    b1h = b1h_ref[...]
    h0 = jnp.zeros((B, HT), jnp.float32)
    h1 = jnp.zeros((B, HT), jnp.float32)
    for t in range(T):
        ballt = jnp.broadcast_to(xb[0:1, 3 * t:3 * t + 3], (B, 3))
        xt = xq[:, 3 * t:3 * t + 3]
        u = jnp.concatenate([ballt, xt], axis=1)
        h0 = _cell(_dot(u, w0i) + b0i, _dot(h0, w0h) + b0h, h0, HT)
        h1 = _cell(_dot(h0, w1i) + b1i, _dot(h1, w1h) + b1h, h1, HT)
    out_ref[...] = h1


def _ta_encode(xq, xb, w0i, w0h, b0i, b0h, w1i, w1h, b1i, b1h):
    return pl.pallas_call(
        _ta_body,
        out_shape=jax.ShapeDtypeStruct((B, HT), jnp.float32),
    )(xq, xb, w0i, w0h, b0i, b0h, w1i, w1h, b1i, b1h)


def _upd_body(p0_ref, p1_ref, h_ref, wih_ref, whh_ref, bih_ref, bhh_ref,
              wmsg_ref, bmsg_ref, h1_ref, m2_ref):
    agg = p0_ref[...] + p1_ref[...]
    h = h_ref[...]
    gi = _dot(agg, wih_ref[...]) + bih_ref[...]
    gh = _dot(h, whh_ref[...]) + bhh_ref[...]
    hn = _cell(gi, gh, h, HG)
    h1_ref[...] = hn
    m2_ref[...] = _dot(hn, wmsg_ref[...]) + bmsg_ref[...]


def _gg_update(p0, p1, h, wih_t, whh_t, bih, bhh, wmsg_t, bmsg):
    nblk = NPAD // BA
    full = lambda s: pl.BlockSpec(s, lambda i: (0, 0))
    blk = pl.BlockSpec((BA, HG), lambda i: (i, 0))
    return pl.pallas_call(
        _upd_body,
        grid=(nblk,),
        in_specs=[blk, blk, blk, full(wih_t.shape), full(whh_t.shape),
                  full(bih.shape), full(bhh.shape), full(wmsg_t.shape),
                  full(bmsg.shape)],
        out_specs=[blk, blk],
        out_shape=[
            jax.ShapeDtypeStruct((NPAD, HG), jnp.float32),
            jax.ShapeDtypeStruct((NPAD, HG), jnp.float32),
        ],
    )(p0, p1, h, wih_t, whh_t, bih, bhh, wmsg_t, bmsg)


def _head_body(pq0_ref, pq1_ref, hq_ref, tf2_ref, wih_ref, whh_ref, bih_ref,
               bhh_ref, w1_ref, b1_ref, w2_ref, b2_ref, out_ref):
    aggq = pq0_ref[...] + pq1_ref[...]
    hq = hq_ref[...]
    gi = _dot(aggq, wih_ref[...]) + bih_ref[...]
    gh = _dot(hq, whh_ref[...]) + bhh_ref[...]
    h2 = _cell(gi, gh, hq, HG)
    w1 = w1_ref[...]
    hid = jax.nn.relu(_dot(h2, w1[:HG, :]) + _dot(tf2_ref[...], w1[HG:, :])
                      + b1_ref[...])
    out_ref[...] = jax.nn.sigmoid(_dot(hid, w2_ref[...]) + b2_ref[...])


def _head(pq0, pq1, hq, tf2, wih_t, whh_t, bih, bhh, w1, b1, w2, b2):
    return pl.pallas_call(
        _head_body,
        out_shape=jax.ShapeDtypeStruct((B, 1), jnp.float32),
    )(pq0, pq1, hq, tf2, wih_t, whh_t, bih, bhh, w1, b1, w2, b2)


def _sc_mesh():
    return plsc.VectorSubcoreMesh(core_axis_name="c", subcore_axis_name="s")


_SC_PARAMS = pltpu.CompilerParams(use_tc_tiling_on_sc=False)


NB = 4


def _scatter_chunks(mspm, src_hbm, dst_hbm, agg, sidx, didx, bufs, gsems,
                    ssems, wid):
    pltpu.sync_copy(src_hbm.at[wid], sidx)
    pltpu.sync_copy(dst_hbm.at[wid], didx)

    def body(j, carry):
        gh = []
        for b in range(NB):
            jj = j * NB + b
            gh.append(pltpu.async_copy(mspm.at[sidx.at[jj]], bufs[b],
                                       gsems[b]))
        sh = []
        for b in range(NB):
            jj = j * NB + b
            gh[b].wait()
            sh.append(pltpu.async_copy(bufs[b], agg.at[didx.at[jj]],
                                       ssems[b], add=True))
        for b in range(NB):
            sh[b].wait()
        return carry

    lax.fori_loop(0, KCH // NB, body, 0)


def _gather_rows(tbl_hbm, q_hbm, out_hbm, qidx, qrows, sem, base, cnt):
    pltpu.sync_copy(q_hbm.at[pl.ds(base, cnt)], qidx)
    pltpu.async_copy(tbl_hbm.at[qidx], qrows, sem).wait()
    pltpu.sync_copy(qrows, out_hbm.at[pl.ds(base, cnt)])


def _sc_step1(m1, srcp, dstp, q, xp, zrows):
    @functools.partial(
        pl.kernel,
        out_type=[
            jax.ShapeDtypeStruct((NPAD, HG), jnp.float32),
            jax.ShapeDtypeStruct((NPAD, HG), jnp.float32),
            jax.ShapeDtypeStruct((B, 64), jnp.float32),
        ],
        mesh=_sc_mesh(),
        scratch_types=[
            pltpu.VMEM((KCH, EC), jnp.int32),
            pltpu.VMEM((KCH, EC), jnp.int32),
            [pltpu.VMEM((EC, HG), jnp.float32)] * NB,
            [pltpu.SemaphoreType.DMA] * NB,
            [pltpu.SemaphoreType.DMA] * NB,
            pltpu.VMEM((QW,), jnp.int32),
            pltpu.VMEM((QW, 64), jnp.float32),
            pltpu.VMEM_SHARED((NPAD, HG), jnp.float32),
            pltpu.SemaphoreType.DMA,
        ],
        compiler_params=_SC_PARAMS,
    )
    def k(m_hbm, src_hbm, dst_hbm, q_hbm, xp_hbm, z_hbm,
          p0_hbm, p1_hbm, xq_hbm,
          sidx, didx, bufs, gsems, ssems, qidx, qrows, agg, sem):
        c = lax.axis_index("c")
        s = lax.axis_index("s")
        wid = s * NC + c
        pltpu.sync_copy(z_hbm.at[pl.ds(s * RPT, RPT)],
                        agg.at[pl.ds(s * RPT, RPT)])
        plsc.subcore_barrier()
        _scatter_chunks(m_hbm, src_hbm, dst_hbm, agg, sidx, didx, bufs,
                        gsems, ssems, wid)
        plsc.subcore_barrier()

        @pl.when(c == 0)
        def _():
            pltpu.sync_copy(agg.at[pl.ds(s * RPT, RPT)],
                            p0_hbm.at[pl.ds(s * RPT, RPT)])

        @pl.when(c == 1)
        def _():
            pltpu.sync_copy(agg.at[pl.ds(s * RPT, RPT)],
                            p1_hbm.at[pl.ds(s * RPT, RPT)])

        _gather_rows(xp_hbm, q_hbm, xq_hbm, qidx, qrows, sem, wid * QW, QW)

    return k(m1, srcp, dstp, q, xp, zrows)


def _sc_step2(m2, srcp, dstp, q, h1, zrows):
    @functools.partial(
        pl.kernel,
        out_type=[
            jax.ShapeDtypeStruct((NPAD, HG), jnp.float32),
            jax.ShapeDtypeStruct((NPAD, HG), jnp.float32),
            jax.ShapeDtypeStruct((B, HG), jnp.float32),
            jax.ShapeDtypeStruct((B, HG), jnp.float32),
            jax.ShapeDtypeStruct((B, HG), jnp.float32),
        ],
        mesh=_sc_mesh(),
        scratch_types=[
            pltpu.VMEM((KCH, EC), jnp.int32),
            pltpu.VMEM((KCH, EC), jnp.int32),
            [pltpu.VMEM((EC, HG), jnp.float32)] * NB,
            [pltpu.SemaphoreType.DMA] * NB,
            [pltpu.SemaphoreType.DMA] * NB,
            pltpu.VMEM((QW,), jnp.int32),
            pltpu.VMEM((QW, HG), jnp.float32),
            pltpu.VMEM((QT,), jnp.int32),
            pltpu.VMEM((QT, HG), jnp.float32),
            pltpu.VMEM_SHARED((NPAD, HG), jnp.float32),
            pltpu.SemaphoreType.DMA,
        ],
        compiler_params=_SC_PARAMS,
    )
    def k(m_hbm, src_hbm, dst_hbm, q_hbm, h1_hbm, z_hbm,
          p0_hbm, p1_hbm, hq_hbm, pq0_hbm, pq1_hbm,
          sidx, didx, bufs, gsems, ssems, qidx, qrows, qidx2, qrows2,
          agg, sem):
        c = lax.axis_index("c")
        s = lax.axis_index("s")
        wid = s * NC + c
        pltpu.sync_copy(z_hbm.at[pl.ds(s * RPT, RPT)],
                        agg.at[pl.ds(s * RPT, RPT)])
        plsc.subcore_barrier()
        _scatter_chunks(m_hbm, src_hbm, dst_hbm, agg, sidx, didx, bufs,
                        gsems, ssems, wid)
        plsc.subcore_barrier()

        @pl.when(c == 0)
        def _():
            pltpu.sync_copy(agg.at[pl.ds(s * RPT, RPT)],
                            p0_hbm.at[pl.ds(s * RPT, RPT)])

        @pl.when(c == 1)
        def _():
            pltpu.sync_copy(agg.at[pl.ds(s * RPT, RPT)],
                            p1_hbm.at[pl.ds(s * RPT, RPT)])

        _gather_rows(h1_hbm, q_hbm, hq_hbm, qidx, qrows, sem, wid * QW, QW)
        plsc.subcore_barrier()
        pltpu.sync_copy(q_hbm.at[pl.ds(s * QT, QT)], qidx2)

        @pl.when(c == 0)
        def _():
            pltpu.async_copy(p0_hbm.at[qidx2], qrows2, sem).wait()
            pltpu.sync_copy(qrows2, pq0_hbm.at[pl.ds(s * QT, QT)])

        @pl.when(c == 1)
        def _():
            pltpu.async_copy(p1_hbm.at[qidx2], qrows2, sem).wait()
            pltpu.sync_copy(qrows2, pq1_hbm.at[pl.ds(s * QT, QT)])

    return k(m2, srcp, dstp, q, h1, zrows)


def kernel(g, x, q_from, na_Wih, na_Whh, na_bih, na_bhh, ta0_Wih, ta0_Whh,
           ta0_bih, ta0_bhh, ta1_Wih, ta1_Whh, ta1_bih, ta1_bhh, gg_Wmsg,
           gg_bmsg, gg_Wih, gg_Whh, gg_bih, gg_bhh, p_W1, p_b1, p_W2, p_b2):
    f32 = jnp.float32
    i32 = jnp.int32

    x_flat = x.reshape(N, T * 3).astype(f32)
    xp = jnp.pad(x_flat, ((0, NPAD - N), (0, 64 - T * 3)))
    xb = xp[0:1]
    padidx = N + jnp.arange(EPAD - E, dtype=i32) % (NPAD - N)
    src = jnp.concatenate([g[0].astype(i32), padidx]).reshape(NW, KCH, EC)
    dst = jnp.concatenate([g[1].astype(i32), padidx]).reshape(NW, KCH, EC)
    q = q_from.astype(i32)
    zrows = jnp.zeros((NPAD, HG), f32)
    row = lambda b: b.reshape(1, -1).astype(f32)

    tf, m1 = _na_encode(xp, na_Wih.T, na_Whh.T, row(na_bih), row(na_bhh),
                        gg_Wmsg.T, row(gg_bmsg))

    p0, p1, xq = _sc_step1(m1, src, dst, q, xp, zrows)

    tf2q = _ta_encode(xq, xb, ta0_Wih.T, ta0_Whh.T, row(ta0_bih),
                      row(ta0_bhh), ta1_Wih.T, ta1_Whh.T, row(ta1_bih),
                      row(ta1_bhh))

    h1, m2 = _gg_update(p0, p1, tf, gg_Wih.T, gg_Whh.T, row(gg_bih),
                        row(gg_bhh), gg_Wmsg.T, row(gg_bmsg))

    _, _, hq, pq0, pq1 = _sc_step2(m2, src, dst, q, h1, zrows)

    return _head(pq0, pq1, hq, tf2q, gg_Wih.T, gg_Whh.T, row(gg_bih),
                 row(gg_bhh), p_W1.astype(f32), row(p_b1),
                 p_W2.astype(f32), row(p_b2))
